# Initial kernel scaffold; baseline (speedup 1.0000x reference)
import jax
import jax.numpy as jnp
from jax.experimental import pallas as pl


def kernel(x, edge_index, edge_weight, W, b):
    raise NotImplementedError("write your pallas kernel here")


# speedup vs baseline: 4.1842x; 4.1842x over previous
import functools

import jax
import jax.numpy as jnp
from jax import lax
from jax.experimental import pallas as pl
from jax.experimental.pallas import tpu as pltpu
from jax.experimental.pallas import tpu_sc as plsc

N_NODES = 10000
N_EDGES = 320000
D = 128

NC = 2
NS = 16
NW = NC * NS
E_PER_W = N_EDGES // NW
SUB = 40
NSUB = 5
E_CHUNK = SUB * NSUB
N_CHUNKS = E_PER_W // E_CHUNK
ROWS_PER_TILE = 632
N_PAD = ROWS_PER_TILE * NS


def _sc_aggregate(xf, src2, dst2, wts):
    mesh = plsc.VectorSubcoreMesh(core_axis_name="c", subcore_axis_name="s")

    @functools.partial(
        pl.kernel,
        out_type=jax.ShapeDtypeStruct((NC, N_PAD, D), jnp.float32),
        mesh=mesh,
        scratch_types=[
            pltpu.VMEM((NSUB, SUB), jnp.int32),
            pltpu.VMEM((NSUB, SUB), jnp.int32),
            pltpu.VMEM((E_CHUNK,), jnp.float32),
            pltpu.VMEM((E_CHUNK, D), jnp.float32),
            pltpu.VMEM_SHARED((N_PAD, D), jnp.float32),
            pltpu.SemaphoreType.DMA,
        ],
    )
    def agg(x_hbm, src_hbm, dst_hbm, w_hbm, out_hbm,
            src_v, dst_v, w_v, rows_v, acc, sem):
        cid = lax.axis_index("c")
        sid = lax.axis_index("s")
        wid = cid * NS + sid

        zero16 = jnp.zeros((16,), jnp.float32)

        def zero_body(i, carry):
            for j in range(D // 16):
                rows_v[i, pl.ds(j * 16, 16)] = zero16
            return carry

        lax.fori_loop(0, E_CHUNK, zero_body, 0)
        base = sid * ROWS_PER_TILE
        nfull = ROWS_PER_TILE // E_CHUNK
        for t in range(nfull):
            pltpu.sync_copy(rows_v, acc.at[pl.ds(base + t * E_CHUNK, E_CHUNK)])
        rem = ROWS_PER_TILE - nfull * E_CHUNK
        if rem:
            pltpu.sync_copy(rows_v.at[pl.ds(0, rem)],
                            acc.at[pl.ds(base + nfull * E_CHUNK, rem)])
        plsc.subcore_barrier()

        def chunk_body(c, carry):
            e0 = wid * E_PER_W + c * E_CHUNK
            for j in range(NSUB):
                pltpu.sync_copy(src_hbm.at[pl.ds(e0 + j * SUB, SUB)],
                                src_v.at[j])
                pltpu.sync_copy(dst_hbm.at[pl.ds(e0 + j * SUB, SUB)],
                                dst_v.at[j])
            pltpu.sync_copy(w_hbm.at[pl.ds(e0, E_CHUNK)], w_v)
            cps = [
                pltpu.async_copy(x_hbm.at[src_v.at[j]],
                                 rows_v.at[pl.ds(j * SUB, SUB)], sem)
                for j in range(NSUB)
            ]
            for cp in cps:
                cp.wait()

            def mul_body(g, cc):
                wv = w_v[pl.ds(g * 16, 16)]
                e0 = g * 16
                for l in range(16):
                    w = wv[l]
                    for j in range(D // 16):
                        sl = pl.ds(j * 16, 16)
                        rows_v[e0 + l, sl] = rows_v[e0 + l, sl] * w
                return cc

            lax.fori_loop(0, E_CHUNK // 16, mul_body, 0)
            tail = E_CHUNK - (E_CHUNK // 16) * 16
            if tail:
                wv = w_v[pl.ds(E_CHUNK - 16, 16)]
                for l in range(16 - tail, 16):
                    e = E_CHUNK - 16 + l
                    w = wv[l]
                    for j in range(D // 16):
                        sl = pl.ds(j * 16, 16)
                        rows_v[e, sl] = rows_v[e, sl] * w
            for j in range(NSUB):
                pltpu.sync_copy(rows_v.at[pl.ds(j * SUB, SUB)],
                                acc.at[dst_v.at[j]], add=True)
            return carry

        lax.fori_loop(0, N_CHUNKS, chunk_body, 0)
        plsc.subcore_barrier()

        pltpu.sync_copy(acc.at[pl.ds(base, ROWS_PER_TILE)],
                        out_hbm.at[cid, pl.ds(base, ROWS_PER_TILE), :])

    return agg(xf, src2, dst2, wts)


def _tc_body(p0_ref, p1_ref, w_ref, b_ref, o_ref):
    s = p0_ref[...] + p1_ref[...]
    y = jnp.dot(s, w_ref[...], preferred_element_type=jnp.float32)
    o_ref[...] = jnp.maximum(y + b_ref[...], 0.0)


BLK = 1000


def _tc_finish(p0, p1, W, b2):
    return pl.pallas_call(
        _tc_body,
        grid=(N_NODES // BLK,),
        in_specs=[
            pl.BlockSpec((BLK, D), lambda i: (i, 0)),
            pl.BlockSpec((BLK, D), lambda i: (i, 0)),
            pl.BlockSpec((D, D), lambda i: (0, 0)),
            pl.BlockSpec((1, D), lambda i: (0, 0)),
        ],
        out_specs=pl.BlockSpec((BLK, D), lambda i: (i, 0)),
        out_shape=jax.ShapeDtypeStruct((N_NODES, D), jnp.float32),
    )(p0, p1, W, b2)


def kernel(x, edge_index, edge_weight, W, b):
    xf = x.reshape(N_NODES, D)
    ei = edge_index.astype(jnp.int32)
    partials = _sc_aggregate(xf, ei[0], ei[1], edge_weight)
    out = _tc_finish(partials[0, :N_NODES], partials[1, :N_NODES],
                     W, b.reshape(1, D))
    return out.reshape(1, N_NODES, D)


# speedup vs baseline: 11.9881x; 2.8651x over previous
import functools

import jax
import jax.numpy as jnp
from jax import lax
from jax.experimental import pallas as pl
from jax.experimental.pallas import tpu as pltpu
from jax.experimental.pallas import tpu_sc as plsc

N_NODES = 10000
N_EDGES = 320000
D = 128

NC = 2
NS = 16
NW = NC * NS
E_PER_W = N_EDGES // NW
E_CHUNK = 80
N_CHUNKS = E_PER_W // E_CHUNK
NBUF = 3
ROWS_PER_TILE = 632
N_PAD = ROWS_PER_TILE * NS


def _sc_aggregate(xf, src1, dst3, wts):
    mesh = plsc.VectorSubcoreMesh(core_axis_name="c", subcore_axis_name="s")

    @functools.partial(
        pl.kernel,
        out_type=jax.ShapeDtypeStruct((NC, N_PAD, D), jnp.float32),
        mesh=mesh,
        scratch_types=[
            pltpu.VMEM((NBUF, E_CHUNK), jnp.int32),
            pltpu.VMEM((NBUF, E_CHUNK), jnp.float32),
            pltpu.VMEM((NBUF * E_CHUNK, D), jnp.float32),
            pltpu.VMEM((N_CHUNKS, E_CHUNK), jnp.int32),
            pltpu.VMEM_SHARED((N_PAD, D), jnp.float32),
            [pltpu.SemaphoreType.DMA for _ in range(NBUF)],
            [pltpu.SemaphoreType.DMA for _ in range(NBUF)],
            [pltpu.SemaphoreType.DMA for _ in range(NBUF)],
        ],
    )
    def agg(x_hbm, src_hbm, dst_hbm, w_hbm, out_hbm,
            src_v, w_v, rows, dst_big, acc, sem_i, sem_g, sem_s):
        cid = lax.axis_index("c")
        sid = lax.axis_index("s")
        wid = cid * NS + sid
        ebase = wid * E_PER_W

        stage_cp = pltpu.async_copy(dst_hbm.at[wid], dst_big, sem_i[0])

        zero16 = jnp.zeros((16,), jnp.float32)

        def zero_body(i, carry):
            for j in range(D // 16):
                rows[i, pl.ds(j * 16, 16)] = zero16
            return carry

        lax.fori_loop(0, E_CHUNK, zero_body, 0)
        base = sid * ROWS_PER_TILE
        zview = rows.at[pl.ds(0, E_CHUNK)]
        nfull = ROWS_PER_TILE // E_CHUNK
        for t in range(nfull):
            pltpu.sync_copy(zview, acc.at[pl.ds(base + t * E_CHUNK, E_CHUNK)])
        rem = ROWS_PER_TILE - nfull * E_CHUNK
        if rem:
            pltpu.sync_copy(rows.at[pl.ds(0, rem)],
                            acc.at[pl.ds(base + nfull * E_CHUNK, rem)])
        stage_cp.wait()
        plsc.subcore_barrier()

        def rows_at(bb):
            return rows.at[pl.ds(bb * E_CHUNK, E_CHUNK)]

        def issue_idx(c, bb):
            pltpu.async_copy(src_hbm.at[pl.ds(ebase + c * E_CHUNK, E_CHUNK)],
                             src_v.at[bb], sem_i[bb])
            pltpu.async_copy(w_hbm.at[pl.ds(ebase + c * E_CHUNK, E_CHUNK)],
                             w_v.at[bb], sem_i[bb])

        def wait_idx(c, bb):
            pltpu.make_async_copy(
                src_hbm.at[pl.ds(ebase + c * E_CHUNK, E_CHUNK)],
                src_v.at[bb], sem_i[bb]).wait()
            pltpu.make_async_copy(
                w_hbm.at[pl.ds(ebase + c * E_CHUNK, E_CHUNK)],
                w_v.at[bb], sem_i[bb]).wait()

        def issue_gather(bb):
            pltpu.async_copy(x_hbm.at[src_v.at[bb]], rows_at(bb), sem_g[bb])

        def wait_gather(bb):
            pltpu.make_async_copy(x_hbm.at[src_v.at[bb]], rows_at(bb),
                                  sem_g[bb]).wait()

        def issue_scatter(c, bb):
            pltpu.async_copy(rows_at(bb), acc.at[dst_big.at[c]], sem_s[bb],
                             add=True)

        def wait_scatter(c, bb):
            pltpu.make_async_copy(rows_at(bb), acc.at[dst_big.at[c]],
                                  sem_s[bb]).wait()

        def multiply(b):
            roff = b * E_CHUNK

            def grp(g, cc):
                wv = w_v[b, pl.ds(g * 16, 16)]
                for l in range(16):
                    w = wv[l]
                    e = roff + g * 16 + l
                    for j in range(D // 16):
                        sl = pl.ds(j * 16, 16)
                        rows[e, sl] = rows[e, sl] * w
                return cc

            lax.fori_loop(0, E_CHUNK // 16, grp, 0)

        issue_idx(0, 0)
        issue_idx(1, 1)
        wait_idx(0, 0)
        issue_gather(0)

        def chunk_body(c, carry):
            b = lax.rem(c, NBUF)

            @pl.when(c + 2 < N_CHUNKS)
            def _():
                b2 = lax.rem(c + 2, NBUF)
                for bb in range(NBUF):
                    @pl.when(b2 == bb)
                    def _(bb=bb):
                        issue_idx(c + 2, bb)

            @pl.when(c + 1 < N_CHUNKS)
            def _():
                b1 = lax.rem(c + 1, NBUF)
                for bb in range(NBUF):
                    @pl.when(b1 == bb)
                    def _(bb=bb):
                        wait_idx(c + 1, bb)

                        @pl.when(c - 2 >= 0)
                        def _():
                            wait_scatter(c - 2, bb)

                        issue_gather(bb)

            for bb in range(NBUF):
                @pl.when(b == bb)
                def _(bb=bb):
                    wait_gather(bb)

            multiply(b)

            for bb in range(NBUF):
                @pl.when(b == bb)
                def _(bb=bb):
                    issue_scatter(c, bb)

            return carry

        lax.fori_loop(0, N_CHUNKS, chunk_body, 0)
        for k in (N_CHUNKS - 3, N_CHUNKS - 2, N_CHUNKS - 1):
            wait_scatter(k, k % NBUF)
        plsc.subcore_barrier()

        pltpu.sync_copy(acc.at[pl.ds(base, ROWS_PER_TILE)],
                        out_hbm.at[cid, pl.ds(base, ROWS_PER_TILE), :])

    return agg(xf, src1, dst3, wts)


def _tc_body(p0_ref, p1_ref, w_ref, b_ref, o_ref):
    s = p0_ref[...] + p1_ref[...]
    y = jnp.dot(s, w_ref[...], preferred_element_type=jnp.float32)
    o_ref[...] = jnp.maximum(y + b_ref[...], 0.0)


BLK = 1000


def _tc_finish(p0, p1, W, b2):
    return pl.pallas_call(
        _tc_body,
        grid=(N_NODES // BLK,),
        in_specs=[
            pl.BlockSpec((BLK, D), lambda i: (i, 0)),
            pl.BlockSpec((BLK, D), lambda i: (i, 0)),
            pl.BlockSpec((D, D), lambda i: (0, 0)),
            pl.BlockSpec((1, D), lambda i: (0, 0)),
        ],
        out_specs=pl.BlockSpec((BLK, D), lambda i: (i, 0)),
        out_shape=jax.ShapeDtypeStruct((N_NODES, D), jnp.float32),
    )(p0, p1, W, b2)


def kernel(x, edge_index, edge_weight, W, b):
    xf = x.reshape(N_NODES, D)
    ei = edge_index.astype(jnp.int32)
    dst3 = ei[1].reshape(NW, N_CHUNKS, E_CHUNK)
    partials = _sc_aggregate(xf, ei[0], dst3, edge_weight)
    out = _tc_finish(partials[0, :N_NODES], partials[1, :N_NODES],
                     W, b.reshape(1, D))
    return out.reshape(1, N_NODES, D)


# speedup vs baseline: 12.4751x; 1.0406x over previous
import functools

import jax
import jax.numpy as jnp
from jax import lax
from jax.experimental import pallas as pl
from jax.experimental.pallas import tpu as pltpu
from jax.experimental.pallas import tpu_sc as plsc

N_NODES = 10000
N_EDGES = 320000
D = 128

NC = 2
NS = 16
NW = NC * NS
E_PER_W = N_EDGES // NW
E_CHUNK = 80
N_CHUNKS = E_PER_W // E_CHUNK
NBUF = 3
ROWS_PER_TILE = 632
N_PAD = ROWS_PER_TILE * NS


def _sc_aggregate(xf, src1, dst3, wts):
    mesh = plsc.VectorSubcoreMesh(core_axis_name="c", subcore_axis_name="s")

    @functools.partial(
        pl.kernel,
        out_type=jax.ShapeDtypeStruct((NC, N_PAD, D), jnp.float32),
        mesh=mesh,
        scratch_types=[
            pltpu.VMEM((NBUF, E_CHUNK), jnp.int32),
            pltpu.VMEM((NBUF, E_CHUNK), jnp.float32),
            pltpu.VMEM((NBUF * E_CHUNK, D), jnp.float32),
            pltpu.VMEM((N_CHUNKS, E_CHUNK), jnp.int32),
            pltpu.VMEM_SHARED((N_PAD, D), jnp.float32),
            [pltpu.SemaphoreType.DMA for _ in range(NBUF)],
            [pltpu.SemaphoreType.DMA for _ in range(NBUF)],
            [pltpu.SemaphoreType.DMA for _ in range(NBUF)],
        ],
    )
    def agg(x_hbm, src_hbm, dst_hbm, w_hbm, out_hbm,
            src_v, w_v, rows, dst_big, acc, sem_i, sem_g, sem_s):
        cid = lax.axis_index("c")
        sid = lax.axis_index("s")
        wid = cid * NS + sid
        ebase = wid * E_PER_W

        stage_cp = pltpu.async_copy(dst_hbm.at[wid], dst_big, sem_i[0])

        zero16 = jnp.zeros((16,), jnp.float32)

        def zero_body(i, carry):
            for j in range(D // 16):
                rows[i, pl.ds(j * 16, 16)] = zero16
            return carry

        lax.fori_loop(0, E_CHUNK, zero_body, 0)
        base = sid * ROWS_PER_TILE
        zview = rows.at[pl.ds(0, E_CHUNK)]
        nfull = ROWS_PER_TILE // E_CHUNK
        for t in range(nfull):
            pltpu.sync_copy(zview, acc.at[pl.ds(base + t * E_CHUNK, E_CHUNK)])
        rem = ROWS_PER_TILE - nfull * E_CHUNK
        if rem:
            pltpu.sync_copy(rows.at[pl.ds(0, rem)],
                            acc.at[pl.ds(base + nfull * E_CHUNK, rem)])
        stage_cp.wait()
        plsc.subcore_barrier()

        def rows_at(bb):
            return rows.at[pl.ds(bb * E_CHUNK, E_CHUNK)]

        def issue_idx(c, bb):
            pltpu.async_copy(src_hbm.at[pl.ds(ebase + c * E_CHUNK, E_CHUNK)],
                             src_v.at[bb], sem_i[bb])
            pltpu.async_copy(w_hbm.at[pl.ds(ebase + c * E_CHUNK, E_CHUNK)],
                             w_v.at[bb], sem_i[bb])

        def wait_idx(c, bb):
            pltpu.make_async_copy(
                src_hbm.at[pl.ds(ebase + c * E_CHUNK, E_CHUNK)],
                src_v.at[bb], sem_i[bb]).wait()
            pltpu.make_async_copy(
                w_hbm.at[pl.ds(ebase + c * E_CHUNK, E_CHUNK)],
                w_v.at[bb], sem_i[bb]).wait()

        def issue_gather(bb):
            pltpu.async_copy(x_hbm.at[src_v.at[bb]], rows_at(bb), sem_g[bb])

        def wait_gather(bb):
            pltpu.make_async_copy(x_hbm.at[src_v.at[bb]], rows_at(bb),
                                  sem_g[bb]).wait()

        def issue_scatter(c, bb):
            pltpu.async_copy(rows_at(bb), acc.at[dst_big.at[c]], sem_s[bb],
                             add=True)

        def wait_scatter(c, bb):
            pltpu.make_async_copy(rows_at(bb), acc.at[dst_big.at[c]],
                                  sem_s[bb]).wait()

        def multiply(b):
            roff = b * E_CHUNK

            def grp(g, cc):
                wv = w_v[b, pl.ds(g * 16, 16)]
                for l in range(16):
                    w = wv[l]
                    e = roff + g * 16 + l
                    for j in range(D // 16):
                        sl = pl.ds(j * 16, 16)
                        rows[e, sl] = rows[e, sl] * w
                return cc

            lax.fori_loop(0, E_CHUNK // 16, grp, 0)

        issue_idx(0, 0)
        issue_idx(1, 1)
        wait_idx(0, 0)
        issue_gather(0)

        def chunk_body(c, carry):
            b = lax.rem(c, NBUF)

            @pl.when(c + 2 < N_CHUNKS)
            def _():
                b2 = lax.rem(c + 2, NBUF)
                for bb in range(NBUF):
                    @pl.when(b2 == bb)
                    def _(bb=bb):
                        issue_idx(c + 2, bb)

            @pl.when(c + 1 < N_CHUNKS)
            def _():
                b1 = lax.rem(c + 1, NBUF)
                for bb in range(NBUF):
                    @pl.when(b1 == bb)
                    def _(bb=bb):
                        wait_idx(c + 1, bb)

                        @pl.when(c - 2 >= 0)
                        def _():
                            wait_scatter(c - 2, bb)

                        issue_gather(bb)

            for bb in range(NBUF):
                @pl.when(b == bb)
                def _(bb=bb):
                    wait_gather(bb)

            multiply(b)

            for bb in range(NBUF):
                @pl.when(b == bb)
                def _(bb=bb):
                    issue_scatter(c, bb)

            return carry

        lax.fori_loop(0, N_CHUNKS, chunk_body, 0)
        for k in (N_CHUNKS - 3, N_CHUNKS - 2, N_CHUNKS - 1):
            wait_scatter(k, k % NBUF)
        plsc.subcore_barrier()

        pltpu.sync_copy(acc.at[pl.ds(base, ROWS_PER_TILE)],
                        out_hbm.at[cid, pl.ds(base, ROWS_PER_TILE), :])

    return agg(xf, src1, dst3, wts)


def _tc_body(p0_ref, p1_ref, w_ref, b_ref, o_ref):
    s = p0_ref[0] + p1_ref[0]
    y = jnp.dot(s, w_ref[...], preferred_element_type=jnp.float32)
    o_ref[...] = jnp.maximum(y + b_ref[...], 0.0)


BLK = 1000


def _tc_finish(partials, W, b2):
    return pl.pallas_call(
        _tc_body,
        grid=(N_NODES // BLK,),
        in_specs=[
            pl.BlockSpec((1, BLK, D), lambda i: (0, i, 0)),
            pl.BlockSpec((1, BLK, D), lambda i: (1, i, 0)),
            pl.BlockSpec((D, D), lambda i: (0, 0)),
            pl.BlockSpec((1, D), lambda i: (0, 0)),
        ],
        out_specs=pl.BlockSpec((BLK, D), lambda i: (i, 0)),
        out_shape=jax.ShapeDtypeStruct((N_NODES, D), jnp.float32),
    )(partials, partials, W, b2)


def kernel(x, edge_index, edge_weight, W, b):
    xf = x.reshape(N_NODES, D)
    ei = edge_index.astype(jnp.int32)
    dst3 = ei[1].reshape(NW, N_CHUNKS, E_CHUNK)
    partials = _sc_aggregate(xf, ei[0], dst3, edge_weight)
    out = _tc_finish(partials, W, b.reshape(1, D))
    return out.reshape(1, N_NODES, D)
